# baseline (speedup 1.0000x reference)
import functools

import jax
import jax.numpy as jnp
from jax import lax
from jax.experimental import pallas as pl
from jax.experimental.pallas import tpu as pltpu
from jax.experimental.pallas import tpu_sc as plsc


_GROUP = 16
_NBUF = 2
_CB = 2048


def _densify_block(x_ref, o_ref):
    xt = x_ref[...].T
    o_ref[:, 0:64] = xt[: _CB // 2]
    o_ref[:, 64:128] = xt[_CB // 2 :]


def _densify(table_t):
    d, v = table_t.shape
    grid = (v + _CB - 1) // _CB
    return pl.pallas_call(
        _densify_block,
        grid=(grid,),
        in_specs=[pl.BlockSpec((d, _CB), lambda i: (0, i))],
        out_specs=pl.BlockSpec((_CB // 2, 128), lambda i: (i, 0)),
        out_shape=jax.ShapeDtypeStruct((grid * _CB // 2, 128), jnp.float32),
    )(table_t)


@functools.partial(jax.jit, static_argnames=("n_workers", "d"))
def _gather_sc(idx2, table_t, n_workers, d):
    dense = _densify(table_t)
    dense = dense.reshape(dense.shape[0] * 2, d)

    b, l = idx2.shape
    mesh = plsc.VectorSubcoreMesh(core_axis_name="c", subcore_axis_name="s")
    nc = mesh.num_cores
    b_per_w = b // n_workers
    n_groups = b_per_w // _GROUP

    @functools.partial(
        pl.kernel,
        out_type=jax.ShapeDtypeStruct((b, l, d), jnp.float32),
        mesh=mesh,
        scratch_types=[
            pltpu.VMEM((b_per_w, l), jnp.int32),
            pltpu.VMEM((_NBUF, _GROUP, l, d), jnp.float32),
            pltpu.SemaphoreType.DMA,
            pltpu.SemaphoreType.DMA,
            pltpu.SemaphoreType.DMA,
        ],
        compiler_params=pltpu.CompilerParams(use_tc_tiling_on_sc=False),
    )
    def k(idx_hbm, table_hbm, out_hbm, idx_v, rows_v, gsem, osem0, osem1):
        wid = lax.axis_index("s") * nc + lax.axis_index("c")
        base = wid * b_per_w
        pltpu.sync_copy(idx_hbm.at[pl.ds(base, b_per_w)], idx_v)
        osems = (osem0, osem1)

        def group(g, _):
            def for_buf(buf):
                @pl.when(g >= _NBUF)
                def _():
                    pltpu.make_async_copy(
                        rows_v.at[buf],
                        out_hbm.at[pl.ds(base + (g - _NBUF) * _GROUP, _GROUP)],
                        osems[buf],
                    ).wait()

                for c in range(_GROUP):
                    pltpu.make_async_copy(
                        table_hbm.at[idx_v.at[g * _GROUP + c]],
                        rows_v.at[buf, c],
                        gsem,
                    ).start()
                for c in range(_GROUP):
                    pltpu.make_async_copy(
                        table_hbm.at[idx_v.at[g * _GROUP + c]],
                        rows_v.at[buf, c],
                        gsem,
                    ).wait()

                pltpu.make_async_copy(
                    rows_v.at[buf],
                    out_hbm.at[pl.ds(base + g * _GROUP, _GROUP)],
                    osems[buf],
                ).start()

            for buf in range(_NBUF):
                pl.when(lax.rem(g, _NBUF) == buf)(lambda bb=buf: for_buf(bb))
            return 0

        lax.fori_loop(0, n_groups, group, 0)

        for t in range(_NBUF):
            g = n_groups - _NBUF + t
            pltpu.make_async_copy(
                rows_v.at[g % _NBUF],
                out_hbm.at[pl.ds(base + g * _GROUP, _GROUP)],
                osems[g % _NBUF],
            ).wait()

    return k(idx2, dense)


def kernel(indices, table):
    b, l = indices.shape
    v, d = table.shape
    info = plsc.get_sparse_core_info()
    n_workers = info.num_cores * info.num_subcores
    assert b % (n_workers * _GROUP) == 0
    idx2 = indices.astype(jnp.int32)
    j = idx2 & (_CB - 1)
    idx2 = idx2 + j - jnp.where(j < _CB // 2, 0, _CB - 1)
    return _gather_sc(idx2, table.T, n_workers, d)


# speedup vs baseline: 1.0476x; 1.0476x over previous
import functools

import jax
import jax.numpy as jnp
from jax import lax
from jax.experimental import pallas as pl
from jax.experimental.pallas import tpu as pltpu
from jax.experimental.pallas import tpu_sc as plsc


_GROUP = 8
_NBUF = 2
_CB = 2048


def _densify_block(x_ref, o_ref):
    o_ref[:, 0:64] = x_ref[...].T


def _densify(table_t):
    d, v = table_t.shape
    grid = (v + _CB - 1) // _CB
    return pl.pallas_call(
        _densify_block,
        grid=(grid,),
        in_specs=[pl.BlockSpec((d, _CB), lambda i: (0, i))],
        out_specs=pl.BlockSpec((_CB, 128), lambda i: (i, 0)),
        out_shape=jax.ShapeDtypeStruct((grid * _CB, 128), jnp.float32),
    )(table_t)


@functools.partial(jax.jit, static_argnames=("n_workers", "d"))
def _gather_sc(idx2, table_t, n_workers, d):
    dense = _densify(table_t)

    b, l = idx2.shape
    mesh = plsc.VectorSubcoreMesh(core_axis_name="c", subcore_axis_name="s")
    nc = mesh.num_cores
    b_per_w = b // n_workers
    n_groups = b_per_w // _GROUP

    @functools.partial(
        pl.kernel,
        out_type=jax.ShapeDtypeStruct((b, l, 128), jnp.float32),
        mesh=mesh,
        scratch_types=[
            pltpu.VMEM((b_per_w, l), jnp.int32),
            pltpu.VMEM((_NBUF, _GROUP, l, 128), jnp.float32),
            pltpu.SemaphoreType.DMA,
            pltpu.SemaphoreType.DMA,
            pltpu.SemaphoreType.DMA,
        ],
        compiler_params=pltpu.CompilerParams(use_tc_tiling_on_sc=True),
    )
    def k(idx_hbm, table_hbm, out_hbm, idx_v, rows_v, gsem, osem0, osem1):
        wid = lax.axis_index("s") * nc + lax.axis_index("c")
        base = wid * b_per_w
        pltpu.sync_copy(idx_hbm.at[pl.ds(base, b_per_w)], idx_v)
        osems = (osem0, osem1)

        def group(g, _):
            def for_buf(buf):
                @pl.when(g >= _NBUF)
                def _():
                    pltpu.make_async_copy(
                        rows_v.at[buf],
                        out_hbm.at[pl.ds(base + (g - _NBUF) * _GROUP, _GROUP)],
                        osems[buf],
                    ).wait()

                for c in range(_GROUP):
                    pltpu.make_async_copy(
                        table_hbm.at[idx_v.at[g * _GROUP + c]],
                        rows_v.at[buf, c],
                        gsem,
                    ).start()
                for c in range(_GROUP):
                    pltpu.make_async_copy(
                        table_hbm.at[idx_v.at[g * _GROUP + c]],
                        rows_v.at[buf, c],
                        gsem,
                    ).wait()

                pltpu.make_async_copy(
                    rows_v.at[buf],
                    out_hbm.at[pl.ds(base + g * _GROUP, _GROUP)],
                    osems[buf],
                ).start()

            for buf in range(_NBUF):
                pl.when(lax.rem(g, _NBUF) == buf)(lambda bb=buf: for_buf(bb))
            return 0

        lax.fori_loop(0, n_groups, group, 0)

        for t in range(_NBUF):
            g = n_groups - _NBUF + t
            pltpu.make_async_copy(
                rows_v.at[g % _NBUF],
                out_hbm.at[pl.ds(base + g * _GROUP, _GROUP)],
                osems[g % _NBUF],
            ).wait()

    return k(idx2, dense)[:, :, :d]


def kernel(indices, table):
    b, l = indices.shape
    v, d = table.shape
    info = plsc.get_sparse_core_info()
    n_workers = info.num_cores * info.num_subcores
    assert b % (n_workers * _GROUP) == 0
    idx2 = indices.astype(jnp.int32)
    return _gather_sc(idx2, table.T, n_workers, d)


# speedup vs baseline: 1.5568x; 1.4861x over previous
import functools

import jax
import jax.numpy as jnp
from jax import lax
from jax.experimental import pallas as pl
from jax.experimental.pallas import tpu as pltpu
from jax.experimental.pallas import tpu_sc as plsc


_GROUP = 8
_NBUF = 2
_CB = 8192


def _densify_block(x_ref, o_ref):
    o_ref[:, 0:64] = x_ref[...].T


def _densify(table_t):
    d, v = table_t.shape
    grid = (v + _CB - 1) // _CB
    return pl.pallas_call(
        _densify_block,
        grid=(grid,),
        in_specs=[pl.BlockSpec((d, _CB), lambda i: (0, i))],
        out_specs=pl.BlockSpec((_CB, 128), lambda i: (i, 0)),
        out_shape=jax.ShapeDtypeStruct((grid * _CB, 128), jnp.float32),
    )(table_t)


@functools.partial(jax.jit, static_argnames=("n_workers", "d"))
def _gather_sc(idx2, table_t, n_workers, d):
    dense = _densify(table_t)

    b, l = idx2.shape
    mesh = plsc.VectorSubcoreMesh(core_axis_name="c", subcore_axis_name="s")
    nc = mesh.num_cores
    b_per_w = b // n_workers
    n_groups = b_per_w // _GROUP

    @functools.partial(
        pl.kernel,
        out_type=jax.ShapeDtypeStruct((b, l, 128), jnp.float32),
        mesh=mesh,
        scratch_types=[
            pltpu.VMEM((b_per_w, l), jnp.int32),
            pltpu.VMEM((_NBUF, _GROUP, l, 128), jnp.float32),
            pltpu.SemaphoreType.DMA,
            pltpu.SemaphoreType.DMA,
            pltpu.SemaphoreType.DMA,
        ],
        compiler_params=pltpu.CompilerParams(use_tc_tiling_on_sc=True),
    )
    def k(idx_hbm, table_hbm, out_hbm, idx_v, rows_v, gsem, osem0, osem1):
        wid = lax.axis_index("s") * nc + lax.axis_index("c")
        base = wid * b_per_w
        pltpu.sync_copy(idx_hbm.at[pl.ds(base, b_per_w)], idx_v)
        osems = (osem0, osem1)

        def group(g, _):
            def for_buf(buf):
                @pl.when(g >= _NBUF)
                def _():
                    pltpu.make_async_copy(
                        rows_v.at[buf],
                        out_hbm.at[pl.ds(base + (g - _NBUF) * _GROUP, _GROUP)],
                        osems[buf],
                    ).wait()

                for c in range(_GROUP):
                    pltpu.make_async_copy(
                        table_hbm.at[idx_v.at[g * _GROUP + c]],
                        rows_v.at[buf, c],
                        gsem,
                    ).start()
                for c in range(_GROUP):
                    pltpu.make_async_copy(
                        table_hbm.at[idx_v.at[g * _GROUP + c]],
                        rows_v.at[buf, c],
                        gsem,
                    ).wait()

                pltpu.make_async_copy(
                    rows_v.at[buf],
                    out_hbm.at[pl.ds(base + g * _GROUP, _GROUP)],
                    osems[buf],
                ).start()

            for buf in range(_NBUF):
                pl.when(lax.rem(g, _NBUF) == buf)(lambda bb=buf: for_buf(bb))
            return 0

        lax.fori_loop(0, n_groups, group, 0)

        for t in range(_NBUF):
            g = n_groups - _NBUF + t
            pltpu.make_async_copy(
                rows_v.at[g % _NBUF],
                out_hbm.at[pl.ds(base + g * _GROUP, _GROUP)],
                osems[g % _NBUF],
            ).wait()

    return k(idx2, dense)[:, :, :d]


def kernel(indices, table):
    b, l = indices.shape
    v, d = table.shape
    info = plsc.get_sparse_core_info()
    n_workers = info.num_cores * info.num_subcores
    assert b % (n_workers * _GROUP) == 0
    idx2 = indices.astype(jnp.int32)
    return _gather_sc(idx2, table.T, n_workers, d)


# speedup vs baseline: 1.6253x; 1.0440x over previous
import functools

import jax
import jax.numpy as jnp
from jax import lax
from jax.experimental import pallas as pl
from jax.experimental.pallas import tpu as pltpu
from jax.experimental.pallas import tpu_sc as plsc


_GROUP = 8
_NBUF = 2
_CB = 16384


def _densify_block(x_ref, o_ref):
    o_ref[:, 0:64] = x_ref[...].T


def _densify(table_t):
    d, v = table_t.shape
    grid = (v + _CB - 1) // _CB
    return pl.pallas_call(
        _densify_block,
        grid=(grid,),
        in_specs=[pl.BlockSpec((d, _CB), lambda i: (0, i))],
        out_specs=pl.BlockSpec((_CB, 128), lambda i: (i, 0)),
        out_shape=jax.ShapeDtypeStruct((grid * _CB, 128), jnp.float32),
    )(table_t)


@functools.partial(jax.jit, static_argnames=("n_workers", "d"))
def _gather_sc(idx2, table_t, n_workers, d):
    dense = _densify(table_t)

    b, l = idx2.shape
    mesh = plsc.VectorSubcoreMesh(core_axis_name="c", subcore_axis_name="s")
    nc = mesh.num_cores
    b_per_w = b // n_workers
    n_groups = b_per_w // _GROUP

    @functools.partial(
        pl.kernel,
        out_type=jax.ShapeDtypeStruct((b, l, 128), jnp.float32),
        mesh=mesh,
        scratch_types=[
            pltpu.VMEM((b_per_w, l), jnp.int32),
            pltpu.VMEM((_NBUF, _GROUP, l, 128), jnp.float32),
            pltpu.SemaphoreType.DMA,
            pltpu.SemaphoreType.DMA,
            pltpu.SemaphoreType.DMA,
        ],
        compiler_params=pltpu.CompilerParams(use_tc_tiling_on_sc=True),
    )
    def k(idx_hbm, table_hbm, out_hbm, idx_v, rows_v, gsem, osem0, osem1):
        wid = lax.axis_index("s") * nc + lax.axis_index("c")
        base = wid * b_per_w
        pltpu.sync_copy(idx_hbm.at[pl.ds(base, b_per_w)], idx_v)
        osems = (osem0, osem1)

        def group(g, _):
            def for_buf(buf):
                @pl.when(g >= _NBUF)
                def _():
                    pltpu.make_async_copy(
                        rows_v.at[buf],
                        out_hbm.at[pl.ds(base + (g - _NBUF) * _GROUP, _GROUP)],
                        osems[buf],
                    ).wait()

                for c in range(_GROUP):
                    pltpu.make_async_copy(
                        table_hbm.at[idx_v.at[g * _GROUP + c]],
                        rows_v.at[buf, c],
                        gsem,
                    ).start()
                for c in range(_GROUP):
                    pltpu.make_async_copy(
                        table_hbm.at[idx_v.at[g * _GROUP + c]],
                        rows_v.at[buf, c],
                        gsem,
                    ).wait()

                pltpu.make_async_copy(
                    rows_v.at[buf],
                    out_hbm.at[pl.ds(base + g * _GROUP, _GROUP)],
                    osems[buf],
                ).start()

            for buf in range(_NBUF):
                pl.when(lax.rem(g, _NBUF) == buf)(lambda bb=buf: for_buf(bb))
            return 0

        lax.fori_loop(0, n_groups, group, 0)

        for t in range(_NBUF):
            g = n_groups - _NBUF + t
            pltpu.make_async_copy(
                rows_v.at[g % _NBUF],
                out_hbm.at[pl.ds(base + g * _GROUP, _GROUP)],
                osems[g % _NBUF],
            ).wait()

    return k(idx2, dense)[:, :, :d]


def kernel(indices, table):
    b, l = indices.shape
    v, d = table.shape
    info = plsc.get_sparse_core_info()
    n_workers = info.num_cores * info.num_subcores
    assert b % (n_workers * _GROUP) == 0
    idx2 = indices.astype(jnp.int32)
    return _gather_sc(idx2, table.T, n_workers, d)


# speedup vs baseline: 1.6495x; 1.0148x over previous
import functools

import jax
import jax.numpy as jnp
from jax import lax
from jax.experimental import pallas as pl
from jax.experimental.pallas import tpu as pltpu
from jax.experimental.pallas import tpu_sc as plsc


_GROUP = 8
_NBUF = 2
_CB = 32768


def _densify_block(x_ref, o_ref):
    o_ref[:, 0:64] = x_ref[...].T


def _densify(table_t):
    d, v = table_t.shape
    grid = (v + _CB - 1) // _CB
    return pl.pallas_call(
        _densify_block,
        grid=(grid,),
        in_specs=[pl.BlockSpec((d, _CB), lambda i: (0, i))],
        out_specs=pl.BlockSpec((_CB, 128), lambda i: (i, 0)),
        out_shape=jax.ShapeDtypeStruct((grid * _CB, 128), jnp.float32),
    )(table_t)


@functools.partial(jax.jit, static_argnames=("n_workers", "d"))
def _gather_sc(idx2, table_t, n_workers, d):
    dense = _densify(table_t)

    b, l = idx2.shape
    mesh = plsc.VectorSubcoreMesh(core_axis_name="c", subcore_axis_name="s")
    nc = mesh.num_cores
    b_per_w = b // n_workers
    n_groups = b_per_w // _GROUP

    @functools.partial(
        pl.kernel,
        out_type=jax.ShapeDtypeStruct((b, l, 128), jnp.float32),
        mesh=mesh,
        scratch_types=[
            pltpu.VMEM((b_per_w, l), jnp.int32),
            pltpu.VMEM((_NBUF, _GROUP, l, 128), jnp.float32),
            pltpu.SemaphoreType.DMA,
            pltpu.SemaphoreType.DMA,
            pltpu.SemaphoreType.DMA,
        ],
        compiler_params=pltpu.CompilerParams(use_tc_tiling_on_sc=True),
    )
    def k(idx_hbm, table_hbm, out_hbm, idx_v, rows_v, gsem, osem0, osem1):
        wid = lax.axis_index("s") * nc + lax.axis_index("c")
        base = wid * b_per_w
        pltpu.sync_copy(idx_hbm.at[pl.ds(base, b_per_w)], idx_v)
        osems = (osem0, osem1)

        def group(g, _):
            def for_buf(buf):
                @pl.when(g >= _NBUF)
                def _():
                    pltpu.make_async_copy(
                        rows_v.at[buf],
                        out_hbm.at[pl.ds(base + (g - _NBUF) * _GROUP, _GROUP)],
                        osems[buf],
                    ).wait()

                for c in range(_GROUP):
                    pltpu.make_async_copy(
                        table_hbm.at[idx_v.at[g * _GROUP + c]],
                        rows_v.at[buf, c],
                        gsem,
                    ).start()
                for c in range(_GROUP):
                    pltpu.make_async_copy(
                        table_hbm.at[idx_v.at[g * _GROUP + c]],
                        rows_v.at[buf, c],
                        gsem,
                    ).wait()

                pltpu.make_async_copy(
                    rows_v.at[buf],
                    out_hbm.at[pl.ds(base + g * _GROUP, _GROUP)],
                    osems[buf],
                ).start()

            for buf in range(_NBUF):
                pl.when(lax.rem(g, _NBUF) == buf)(lambda bb=buf: for_buf(bb))
            return 0

        lax.fori_loop(0, n_groups, group, 0)

        for t in range(_NBUF):
            g = n_groups - _NBUF + t
            pltpu.make_async_copy(
                rows_v.at[g % _NBUF],
                out_hbm.at[pl.ds(base + g * _GROUP, _GROUP)],
                osems[g % _NBUF],
            ).wait()

    return k(idx2, dense)[:, :, :d]


def kernel(indices, table):
    b, l = indices.shape
    v, d = table.shape
    info = plsc.get_sparse_core_info()
    n_workers = info.num_cores * info.num_subcores
    assert b % (n_workers * _GROUP) == 0
    idx2 = indices.astype(jnp.int32)
    return _gather_sc(idx2, table.T, n_workers, d)
